# Initial kernel scaffold; baseline (speedup 1.0000x reference)
import jax
import jax.numpy as jnp
from jax.experimental import pallas as pl


def kernel(x, codebooks):
    raise NotImplementedError("write your pallas kernel here")


# speedup vs baseline: 1.2065x; 1.2065x over previous
import jax
import jax.numpy as jnp
from jax import lax
from jax.experimental import pallas as pl

NUM_Q = 8
TB = 512


def _rvq_body(x_ref, cb_ref, cbt_ref, quant_ref, idx_ref):
    res = x_ref[...]
    tb, d = res.shape
    num_q, k, _ = cb_ref.shape
    quant = jnp.zeros_like(res)
    kiota = lax.broadcasted_iota(jnp.int32, (tb, k), 1)
    for i in range(num_q):
        w = cb_ref[i]
        wt = cbt_ref[i]
        scores = lax.dot_general(
            res, wt, (((1,), (0,)), ((), ())),
            preferred_element_type=jnp.float32)
        xsq = jnp.sum(res * res, axis=1, keepdims=True)
        wsq = jnp.sum(wt * wt, axis=0, keepdims=True)
        dist = (xsq + wsq) - 2.0 * scores
        m = jnp.min(dist, axis=1, keepdims=True)
        idx = jnp.min(jnp.where(dist == m, kiota, k), axis=1)
        onehot = (kiota == idx[:, None]).astype(jnp.float32)
        q = lax.dot_general(
            onehot, w, (((1,), (0,)), ((), ())),
            preferred_element_type=jnp.float32,
            precision=lax.Precision.HIGHEST)
        q_st = res + (q - res)
        quant = quant + q_st
        res = res - q_st
        idx_ref[i, :] = idx
    quant_ref[...] = quant


def kernel(x, codebooks):
    b, s, d = x.shape
    num_q, k, _ = codebooks.shape
    tokens = b * s
    x2 = x.reshape(tokens, d)
    cbt = jnp.swapaxes(codebooks, 1, 2)

    grid = tokens // TB
    quant, idx_all = pl.pallas_call(
        _rvq_body,
        grid=(grid,),
        in_specs=[
            pl.BlockSpec((TB, d), lambda t: (t, 0)),
            pl.BlockSpec((num_q, k, d), lambda t: (0, 0, 0)),
            pl.BlockSpec((num_q, d, k), lambda t: (0, 0, 0)),
        ],
        out_specs=[
            pl.BlockSpec((TB, d), lambda t: (t, 0)),
            pl.BlockSpec((num_q, TB), lambda t: (0, t)),
        ],
        out_shape=[
            jax.ShapeDtypeStruct((tokens, d), jnp.float32),
            jax.ShapeDtypeStruct((num_q, tokens), jnp.int32),
        ],
    )(x2, codebooks, cbt)

    quantized = quant.reshape(b, s, d)
    indices = idx_all.T.reshape(b, s, num_q)

    total_commitment_loss = jnp.float32(0.0)
    codebook_usage = jnp.zeros((num_q, k), dtype=jnp.float32)
    usage = codebook_usage / (jnp.sum(codebook_usage, axis=1, keepdims=True) + 1e-05)
    entropy = -jnp.sum(usage * jnp.log(usage + 1e-10), axis=1)
    max_entropy = jnp.log(jnp.float32(k))
    diversity_loss = 1.0 - jnp.mean(entropy) / max_entropy
    total_vq_loss = total_commitment_loss + 0.1 * diversity_loss
    return (quantized, indices, total_commitment_loss, diversity_loss, total_vq_loss)


# speedup vs baseline: 2.1645x; 1.7941x over previous
import jax
import jax.numpy as jnp
from jax import lax
from jax.experimental import pallas as pl

NUM_Q = 8
TB = 512


def _rvq_body(x_ref, cb_hi_ref, cb_lo_ref, cbt_ref, quant_ref, idx_ref):
    res = x_ref[...]
    tb, d = res.shape
    num_q, _, k = cbt_ref.shape
    quant = jnp.zeros_like(res)
    kiota = lax.broadcasted_iota(jnp.int32, (tb, k), 1)
    for i in range(num_q):
        wt = cbt_ref[i]
        scores = lax.dot_general(
            res, wt, (((1,), (0,)), ((), ())),
            preferred_element_type=jnp.float32)
        xsq = jnp.sum(res * res, axis=1, keepdims=True)
        wsq = jnp.sum(wt * wt, axis=0, keepdims=True)
        dist = (xsq + wsq) - 2.0 * scores
        m = jnp.min(dist, axis=1, keepdims=True)
        idx = jnp.min(jnp.where(dist == m, kiota, k), axis=1)
        onehot = (kiota == idx[:, None]).astype(jnp.bfloat16)
        q_hi = lax.dot_general(
            onehot, cb_hi_ref[i], (((1,), (0,)), ((), ())),
            preferred_element_type=jnp.float32)
        q_lo = lax.dot_general(
            onehot, cb_lo_ref[i], (((1,), (0,)), ((), ())),
            preferred_element_type=jnp.float32)
        q = q_hi + q_lo
        q_st = res + (q - res)
        quant = quant + q_st
        res = res - q_st
        idx_ref[i, :] = idx
    quant_ref[...] = quant


def kernel(x, codebooks):
    b, s, d = x.shape
    num_q, k, _ = codebooks.shape
    tokens = b * s
    x2 = x.reshape(tokens, d)
    cbt = jnp.swapaxes(codebooks, 1, 2)
    cb_hi = codebooks.astype(jnp.bfloat16)
    cb_lo = (codebooks - cb_hi.astype(jnp.float32)).astype(jnp.bfloat16)

    grid = tokens // TB
    quant, idx_all = pl.pallas_call(
        _rvq_body,
        grid=(grid,),
        in_specs=[
            pl.BlockSpec((TB, d), lambda t: (t, 0)),
            pl.BlockSpec((num_q, k, d), lambda t: (0, 0, 0)),
            pl.BlockSpec((num_q, k, d), lambda t: (0, 0, 0)),
            pl.BlockSpec((num_q, d, k), lambda t: (0, 0, 0)),
        ],
        out_specs=[
            pl.BlockSpec((TB, d), lambda t: (t, 0)),
            pl.BlockSpec((num_q, TB), lambda t: (0, t)),
        ],
        out_shape=[
            jax.ShapeDtypeStruct((tokens, d), jnp.float32),
            jax.ShapeDtypeStruct((num_q, tokens), jnp.int32),
        ],
    )(x2, cb_hi, cb_lo, cbt)

    quantized = quant.reshape(b, s, d)
    indices = idx_all.T.reshape(b, s, num_q)

    total_commitment_loss = jnp.float32(0.0)
    codebook_usage = jnp.zeros((num_q, k), dtype=jnp.float32)
    usage = codebook_usage / (jnp.sum(codebook_usage, axis=1, keepdims=True) + 1e-05)
    entropy = -jnp.sum(usage * jnp.log(usage + 1e-10), axis=1)
    max_entropy = jnp.log(jnp.float32(k))
    diversity_loss = 1.0 - jnp.mean(entropy) / max_entropy
    total_vq_loss = total_commitment_loss + 0.1 * diversity_loss
    return (quantized, indices, total_commitment_loss, diversity_loss, total_vq_loss)
